# Initial kernel scaffold; baseline (speedup 1.0000x reference)
import jax
import jax.numpy as jnp
from jax.experimental import pallas as pl


def kernel(x, edge_index, root_node_idx, batch, W1l, b1, W1r, W2l, b2, W2r, W3l, b3, W3r, W4l, b4, W4r, Wc, bc):
    raise NotImplementedError("write your pallas kernel here")


# speedup vs baseline: 4.4428x; 4.4428x over previous
import functools

import jax
import jax.numpy as jnp
from jax import lax
from jax.experimental import pallas as pl
from jax.experimental.pallas import tpu as pltpu
from jax.experimental.pallas import tpu_sc as plsc

NC = 2
NS = 16
CH = 80
ZR = 40
DW = 128
_R = 2048


def _sc_agg(h, src, dst):
  N, D = h.shape
  E = src.shape[0]
  assert E % (NC * NS) == 0
  epw = E // (NC * NS)
  assert epw % CH == 0 and epw % 8 == 0
  n_chunks = epw // CH
  assert N % NS == 0
  stripe = N // NS
  assert stripe % ZR == 0 and stripe % 8 == 0

  mesh = plsc.VectorSubcoreMesh(core_axis_name="c", subcore_axis_name="s")

  @functools.partial(
      pl.kernel, mesh=mesh,
      out_type=jax.ShapeDtypeStruct((NC, N, D), jnp.float32),
      scratch_types=[
          pltpu.VMEM_SHARED((N, D), jnp.float32),
          pltpu.VMEM((CH,), jnp.int32),
          pltpu.VMEM((CH,), jnp.int32),
          pltpu.VMEM((CH, D), jnp.float32),
          pltpu.VMEM((ZR, D), jnp.float32),
      ])
  def agg_kernel(h_hbm, src_hbm, dst_hbm, p_hbm,
                 acc, idx_s, idx_d, rows, zb):
    c = lax.axis_index("c")
    s = lax.axis_index("s")
    wid = s * NC + c

    @pl.loop(0, ZR)
    def _(i):
      @pl.loop(0, D, step=16)
      def _(j):
        zb.at[i, pl.ds(j, 16)][...] = jnp.zeros((16,), jnp.float32)

    @pl.loop(0, stripe // ZR)
    def _(t):
      pltpu.sync_copy(zb, acc.at[pl.ds(s * stripe + t * ZR, ZR)])

    plsc.subcore_barrier()

    base = wid * epw

    @pl.loop(0, n_chunks)
    def _(k):
      off = base + k * CH
      pltpu.sync_copy(src_hbm.at[pl.ds(off, CH)], idx_s)
      pltpu.sync_copy(dst_hbm.at[pl.ds(off, CH)], idx_d)
      pltpu.sync_copy(h_hbm.at[idx_s], rows)
      pltpu.sync_copy(rows, acc.at[idx_d], add=True)

    plsc.subcore_barrier()

    sl = pl.ds(s * stripe, stripe)
    pltpu.sync_copy(acc.at[sl], p_hbm.at[c, sl])

  return agg_kernel(h, src, dst)


def _sc_deg(dst, N):
  E = dst.shape[0]
  epw = E // (NC * NS)
  n_chunks = epw // CH
  stripe = N // NS

  mesh = plsc.VectorSubcoreMesh(core_axis_name="c", subcore_axis_name="s")

  @functools.partial(
      pl.kernel, mesh=mesh,
      out_type=jax.ShapeDtypeStruct((NC, N, DW), jnp.float32),
      scratch_types=[
          pltpu.VMEM_SHARED((N, DW), jnp.float32),
          pltpu.VMEM((CH,), jnp.int32),
          pltpu.VMEM((CH, DW), jnp.float32),
          pltpu.VMEM((ZR, DW), jnp.float32),
      ])
  def deg_kernel(dst_hbm, d_hbm, dacc, idx_d, ones, zb):
    c = lax.axis_index("c")
    s = lax.axis_index("s")
    wid = s * NC + c

    @pl.loop(0, ZR)
    def _(i):
      @pl.loop(0, DW, step=16)
      def _(j):
        zb.at[i, pl.ds(j, 16)][...] = jnp.zeros((16,), jnp.float32)

    @pl.loop(0, CH)
    def _(i):
      @pl.loop(0, DW, step=16)
      def _(j):
        ones.at[i, pl.ds(j, 16)][...] = jnp.full((16,), 1.0, jnp.float32)

    @pl.loop(0, stripe // ZR)
    def _(t):
      pltpu.sync_copy(zb, dacc.at[pl.ds(s * stripe + t * ZR, ZR)])

    plsc.subcore_barrier()

    base = wid * epw

    @pl.loop(0, n_chunks)
    def _(k):
      off = base + k * CH
      pltpu.sync_copy(dst_hbm.at[pl.ds(off, CH)], idx_d)
      pltpu.sync_copy(ones, dacc.at[idx_d], add=True)

    plsc.subcore_barrier()

    sl = pl.ds(s * stripe, stripe)
    pltpu.sync_copy(dacc.at[sl], d_hbm.at[c, sl])

  return deg_kernel(dst)


def _tc_layer(p0, p1, dscale0, dscale1, h, Wl, Wr, b, first):
  N, D = h.shape
  H = Wl.shape[1]
  grid = (N // _R,)

  def body(p0_r, p1_r, d0_r, d1_r, h_r, wl_r, wr_r, b_r, *outs):
    if first:
      o_r, dinv_r = outs
      deg = (d0_r[0] + d1_r[0])[:, 0:1]
      dinv = 1.0 / jnp.maximum(deg, 1.0)
      dinv_r[...] = dinv
    else:
      o_r, = outs
      dinv = d0_r[...]
    mean = (p0_r[0] + p1_r[0]) * dinv
    z = (jnp.dot(mean, wl_r[...], preferred_element_type=jnp.float32)
         + jnp.dot(h_r[...], wr_r[...], preferred_element_type=jnp.float32)
         + b_r[...])
    o_r[...] = jnp.maximum(z, 0.0)

  row = lambda i: (i, 0)
  fix = lambda i: (0, 0)
  lo3 = lambda i: (0, i, 0)
  hi3 = lambda i: (1, i, 0)
  if first:
    d_specs = [pl.BlockSpec((1, _R, DW), lo3), pl.BlockSpec((1, _R, DW), hi3)]
  else:
    d_specs = [pl.BlockSpec((_R, 1), row), pl.BlockSpec((_R, 1), row)]
  out_specs = [pl.BlockSpec((_R, H), row)]
  out_shape = [jax.ShapeDtypeStruct((N, H), jnp.float32)]
  if first:
    out_specs.append(pl.BlockSpec((_R, 1), row))
    out_shape.append(jax.ShapeDtypeStruct((N, 1), jnp.float32))
  res = pl.pallas_call(
      body,
      grid=grid,
      in_specs=[
          pl.BlockSpec((1, _R, D), lo3),
          pl.BlockSpec((1, _R, D), hi3),
          d_specs[0],
          d_specs[1],
          pl.BlockSpec((_R, D), row),
          pl.BlockSpec((D, H), fix),
          pl.BlockSpec((D, H), fix),
          pl.BlockSpec((1, H), fix),
      ],
      out_specs=out_specs,
      out_shape=out_shape,
  )(p0, p1, dscale0, dscale1, h, Wl, Wr, b)
  return res


def _tc_final(h, batch3d, root2d, Wc1, Wc2, bc):
  N, H = h.shape
  G = root2d.shape[0]
  OUT = Wc1.shape[1]
  nblk = N // _R

  def body(h_r, b_r, r_r, wc1_r, wc2_r, bc_r, o_r, racc, sacc, cacc):
    i = pl.program_id(0)

    @pl.when(i == 0)
    def _():
      racc[...] = jnp.zeros_like(racc)
      sacc[...] = jnp.zeros_like(sacc)
      cacc[...] = jnp.zeros_like(cacc)

    rows = lax.broadcasted_iota(jnp.int32, (G, _R), 1) + i * _R
    rmask = (r_r[...] == rows).astype(jnp.float32)
    g_iota = lax.broadcasted_iota(jnp.int32, (G, _R), 0)
    bmask = (b_r[0] == g_iota).astype(jnp.float32)

    hb = h_r[...]
    racc[...] += jnp.dot(rmask, hb, preferred_element_type=jnp.float32)
    sacc[...] += jnp.dot(bmask, hb, preferred_element_type=jnp.float32)
    cacc[...] = cacc[...] + jnp.sum(bmask, axis=1, keepdims=True)

    @pl.when(i == nblk - 1)
    def _():
      ge = sacc[...] / jnp.maximum(cacc[...], 1.0)
      o_r[...] = (jnp.dot(racc[...], wc1_r[...],
                          preferred_element_type=jnp.float32)
                  + jnp.dot(ge, wc2_r[...],
                            preferred_element_type=jnp.float32)
                  + bc_r[...])

  return pl.pallas_call(
      body,
      grid=(nblk,),
      in_specs=[
          pl.BlockSpec((_R, H), lambda i: (i, 0)),
          pl.BlockSpec((1, 1, _R), lambda i: (i, 0, 0)),
          pl.BlockSpec((G, 1), lambda i: (0, 0)),
          pl.BlockSpec((H, OUT), lambda i: (0, 0)),
          pl.BlockSpec((H, OUT), lambda i: (0, 0)),
          pl.BlockSpec((1, OUT), lambda i: (0, 0)),
      ],
      out_specs=pl.BlockSpec((G, OUT), lambda i: (0, 0)),
      out_shape=jax.ShapeDtypeStruct((G, OUT), jnp.float32),
      scratch_shapes=[
          pltpu.VMEM((G, H), jnp.float32),
          pltpu.VMEM((G, H), jnp.float32),
          pltpu.VMEM((G, 128), jnp.float32),
      ],
  )(h, batch3d, root2d, Wc1, Wc2, bc)


def kernel(x, edge_index, root_node_idx, batch,
           W1l, b1, W1r, W2l, b2, W2r, W3l, b3, W3r, W4l, b4, W4r, Wc, bc):
  N, D = x.shape
  H = W1l.shape[1]
  G = root_node_idx.shape[0]
  src = edge_index[0]
  dst = edge_index[1]

  PN = ((N + NS * ZR - 1) // (NS * ZR)) * (NS * ZR)
  PN = ((PN + _R - 1) // _R) * _R
  x = jnp.pad(x, ((0, PN - N), (0, 0)))
  batch = jnp.pad(batch, (0, PN - N), constant_values=G)

  d = _sc_deg(dst, PN)
  p = _sc_agg(x, src, dst)
  h, dinv = _tc_layer(p, p, d, d, x, W1l, W1r, b1.reshape(1, H),
                      first=True)
  for Wl, b, Wr in ((W2l, b2, W2r), (W3l, b3, W3r), (W4l, b4, W4r)):
    p = _sc_agg(h, src, dst)
    h, = _tc_layer(p, p, dinv, dinv, h, Wl, Wr, b.reshape(1, H),
                   first=False)
  batch3d = batch.reshape(PN // _R, 1, _R)
  root2d = root_node_idx.reshape(G, 1)
  return _tc_final(h, batch3d, root2d, Wc[:H], Wc[H:], bc.reshape(1, -1))


# speedup vs baseline: 7.7355x; 1.7411x over previous
import functools

import jax
import jax.numpy as jnp
from jax import lax
from jax.experimental import pallas as pl
from jax.experimental.pallas import tpu as pltpu
from jax.experimental.pallas import tpu_sc as plsc

NC = 2
NS = 16
CH = 80
ZR = 40
DW = 128
_R = 2048


def _sc_agg(h, src, dst):
  N, D = h.shape
  E = src.shape[0]
  assert E % (NC * NS) == 0
  epw = E // (NC * NS)
  assert epw % CH == 0 and epw % 8 == 0
  n_chunks = epw // CH
  assert N % NS == 0
  stripe = N // NS
  assert stripe % ZR == 0 and stripe % 8 == 0

  nq, tail = divmod(n_chunks, 4)
  mesh = plsc.VectorSubcoreMesh(core_axis_name="c", subcore_axis_name="s")

  @functools.partial(
      pl.kernel, mesh=mesh,
      out_type=jax.ShapeDtypeStruct((NC, N, D), jnp.float32),
      scratch_types=[
          pltpu.VMEM_SHARED((N, D), jnp.float32),
          pltpu.VMEM((ZR, D), jnp.float32),
      ] + [pltpu.VMEM((CH,), jnp.int32)] * 8
        + [pltpu.VMEM((CH, D), jnp.float32)] * 2
        + [pltpu.SemaphoreType.DMA] * 8)
  def agg_kernel(h_hbm, src_hbm, dst_hbm, p_hbm, acc, zb,
                 sb0, sb1, sb2, sb3, db0, db1, db2, db3, r0, r1,
                 sl0, sl1, sl2, sl3, sg0, sg1, ss0, ss1):
    sbs = (sb0, sb1, sb2, sb3)
    dbs = (db0, db1, db2, db3)
    rows = (r0, r1)
    sls = (sl0, sl1, sl2, sl3)
    sgs = (sg0, sg1)
    sss = (ss0, ss1)
    c = lax.axis_index("c")
    s = lax.axis_index("s")
    wid = s * NC + c

    @pl.loop(0, ZR)
    def _(i):
      @pl.loop(0, D, step=16)
      def _(j):
        zb.at[i, pl.ds(j, 16)][...] = jnp.zeros((16,), jnp.float32)

    @pl.loop(0, stripe // ZR)
    def _(t):
      pltpu.sync_copy(zb, acc.at[pl.ds(s * stripe + t * ZR, ZR)])

    plsc.subcore_barrier()

    base = wid * epw

    def start_idx_load(k, j):
      off = base + k * CH
      pltpu.async_copy(src_hbm.at[pl.ds(off, CH)], sbs[j], sls[j])
      pltpu.async_copy(dst_hbm.at[pl.ds(off, CH)], dbs[j], sls[j])

    def wait_idx(j):
      pltpu.make_async_copy(src_hbm.at[pl.ds(0, CH)], sbs[j], sls[j]).wait()
      pltpu.make_async_copy(dst_hbm.at[pl.ds(0, CH)], dbs[j], sls[j]).wait()

    def wait_scatter(b2, jprev):
      pltpu.make_async_copy(rows[b2], acc.at[dbs[jprev]], sss[b2]).wait()

    def chunk(k, j, kstat):
      b2 = j % 2
      jn = (j + 2) % 4
      if kstat is None:
        @pl.when(k >= 2)
        def _():
          wait_scatter(b2, jn)

        @pl.when(k + 2 < n_chunks)
        def _():
          start_idx_load(k + 2, jn)
      else:
        if kstat >= 2:
          wait_scatter(b2, jn)
        if kstat + 2 < n_chunks:
          start_idx_load(k + 2, jn)
      wait_idx(j)
      pltpu.async_copy(h_hbm.at[sbs[j]], rows[b2], sgs[b2])
      pltpu.make_async_copy(h_hbm.at[sbs[j]], rows[b2], sgs[b2]).wait()
      pltpu.async_copy(rows[b2], acc.at[dbs[j]], sss[b2], add=True)

    start_idx_load(0, 0)
    start_idx_load(1, 1)

    @pl.loop(0, nq)
    def _(q):
      k0 = q * 4
      for j in range(4):
        chunk(k0 + j, j, None)

    for t in range(tail):
      k = nq * 4 + t
      chunk(k, k % 4, k)

    if n_chunks >= 2:
      wait_scatter((n_chunks - 2) % 2, (n_chunks - 2) % 4)
    wait_scatter((n_chunks - 1) % 2, (n_chunks - 1) % 4)

    plsc.subcore_barrier()

    sl = pl.ds(s * stripe, stripe)
    pltpu.sync_copy(acc.at[sl], p_hbm.at[c, sl])

  return agg_kernel(h, src, dst)


def _sc_deg(dst, N):
  E = dst.shape[0]
  epw = E // (NC * NS)
  n_chunks = epw // CH
  stripe = N // NS

  mesh = plsc.VectorSubcoreMesh(core_axis_name="c", subcore_axis_name="s")

  @functools.partial(
      pl.kernel, mesh=mesh,
      out_type=jax.ShapeDtypeStruct((NC, N, DW), jnp.float32),
      scratch_types=[
          pltpu.VMEM_SHARED((N, DW), jnp.float32),
          pltpu.VMEM((CH,), jnp.int32),
          pltpu.VMEM((CH, DW), jnp.float32),
          pltpu.VMEM((ZR, DW), jnp.float32),
      ])
  def deg_kernel(dst_hbm, d_hbm, dacc, idx_d, ones, zb):
    c = lax.axis_index("c")
    s = lax.axis_index("s")
    wid = s * NC + c

    @pl.loop(0, ZR)
    def _(i):
      @pl.loop(0, DW, step=16)
      def _(j):
        zb.at[i, pl.ds(j, 16)][...] = jnp.zeros((16,), jnp.float32)

    @pl.loop(0, CH)
    def _(i):
      @pl.loop(0, DW, step=16)
      def _(j):
        ones.at[i, pl.ds(j, 16)][...] = jnp.full((16,), 1.0, jnp.float32)

    @pl.loop(0, stripe // ZR)
    def _(t):
      pltpu.sync_copy(zb, dacc.at[pl.ds(s * stripe + t * ZR, ZR)])

    plsc.subcore_barrier()

    base = wid * epw

    @pl.loop(0, n_chunks)
    def _(k):
      off = base + k * CH
      pltpu.sync_copy(dst_hbm.at[pl.ds(off, CH)], idx_d)
      pltpu.sync_copy(ones, dacc.at[idx_d], add=True)

    plsc.subcore_barrier()

    sl = pl.ds(s * stripe, stripe)
    pltpu.sync_copy(dacc.at[sl], d_hbm.at[c, sl])

  return deg_kernel(dst)


def _tc_layer(p0, p1, dscale0, dscale1, h, Wl, Wr, b, first):
  N, D = h.shape
  H = Wl.shape[1]
  grid = (N // _R,)

  def body(p0_r, p1_r, d0_r, d1_r, h_r, wl_r, wr_r, b_r, *outs):
    if first:
      o_r, dinv_r = outs
      deg = (d0_r[0] + d1_r[0])[:, 0:1]
      dinv = 1.0 / jnp.maximum(deg, 1.0)
      dinv_r[...] = dinv
    else:
      o_r, = outs
      dinv = d0_r[...]
    mean = (p0_r[0] + p1_r[0]) * dinv
    z = (jnp.dot(mean, wl_r[...], preferred_element_type=jnp.float32)
         + jnp.dot(h_r[...], wr_r[...], preferred_element_type=jnp.float32)
         + b_r[...])
    o_r[...] = jnp.maximum(z, 0.0)

  row = lambda i: (i, 0)
  fix = lambda i: (0, 0)
  lo3 = lambda i: (0, i, 0)
  hi3 = lambda i: (1, i, 0)
  if first:
    d_specs = [pl.BlockSpec((1, _R, DW), lo3), pl.BlockSpec((1, _R, DW), hi3)]
  else:
    d_specs = [pl.BlockSpec((_R, 1), row), pl.BlockSpec((_R, 1), row)]
  out_specs = [pl.BlockSpec((_R, H), row)]
  out_shape = [jax.ShapeDtypeStruct((N, H), jnp.float32)]
  if first:
    out_specs.append(pl.BlockSpec((_R, 1), row))
    out_shape.append(jax.ShapeDtypeStruct((N, 1), jnp.float32))
  res = pl.pallas_call(
      body,
      grid=grid,
      in_specs=[
          pl.BlockSpec((1, _R, D), lo3),
          pl.BlockSpec((1, _R, D), hi3),
          d_specs[0],
          d_specs[1],
          pl.BlockSpec((_R, D), row),
          pl.BlockSpec((D, H), fix),
          pl.BlockSpec((D, H), fix),
          pl.BlockSpec((1, H), fix),
      ],
      out_specs=out_specs,
      out_shape=out_shape,
  )(p0, p1, dscale0, dscale1, h, Wl, Wr, b)
  return res


def _tc_final(h, batch3d, root2d, Wc1, Wc2, bc):
  N, H = h.shape
  G = root2d.shape[0]
  OUT = Wc1.shape[1]
  nblk = N // _R

  def body(h_r, b_r, r_r, wc1_r, wc2_r, bc_r, o_r, racc, sacc, cacc):
    i = pl.program_id(0)

    @pl.when(i == 0)
    def _():
      racc[...] = jnp.zeros_like(racc)
      sacc[...] = jnp.zeros_like(sacc)
      cacc[...] = jnp.zeros_like(cacc)

    rows = lax.broadcasted_iota(jnp.int32, (G, _R), 1) + i * _R
    rmask = (r_r[...] == rows).astype(jnp.float32)
    g_iota = lax.broadcasted_iota(jnp.int32, (G, _R), 0)
    bmask = (b_r[0] == g_iota).astype(jnp.float32)

    hb = h_r[...]
    racc[...] += jnp.dot(rmask, hb, preferred_element_type=jnp.float32)
    sacc[...] += jnp.dot(bmask, hb, preferred_element_type=jnp.float32)
    cacc[...] = cacc[...] + jnp.sum(bmask, axis=1, keepdims=True)

    @pl.when(i == nblk - 1)
    def _():
      ge = sacc[...] / jnp.maximum(cacc[...], 1.0)
      o_r[...] = (jnp.dot(racc[...], wc1_r[...],
                          preferred_element_type=jnp.float32)
                  + jnp.dot(ge, wc2_r[...],
                            preferred_element_type=jnp.float32)
                  + bc_r[...])

  return pl.pallas_call(
      body,
      grid=(nblk,),
      in_specs=[
          pl.BlockSpec((_R, H), lambda i: (i, 0)),
          pl.BlockSpec((1, 1, _R), lambda i: (i, 0, 0)),
          pl.BlockSpec((G, 1), lambda i: (0, 0)),
          pl.BlockSpec((H, OUT), lambda i: (0, 0)),
          pl.BlockSpec((H, OUT), lambda i: (0, 0)),
          pl.BlockSpec((1, OUT), lambda i: (0, 0)),
      ],
      out_specs=pl.BlockSpec((G, OUT), lambda i: (0, 0)),
      out_shape=jax.ShapeDtypeStruct((G, OUT), jnp.float32),
      scratch_shapes=[
          pltpu.VMEM((G, H), jnp.float32),
          pltpu.VMEM((G, H), jnp.float32),
          pltpu.VMEM((G, 128), jnp.float32),
      ],
  )(h, batch3d, root2d, Wc1, Wc2, bc)


def kernel(x, edge_index, root_node_idx, batch,
           W1l, b1, W1r, W2l, b2, W2r, W3l, b3, W3r, W4l, b4, W4r, Wc, bc):
  N, D = x.shape
  H = W1l.shape[1]
  G = root_node_idx.shape[0]
  src = edge_index[0]
  dst = edge_index[1]

  PN = ((N + NS * ZR - 1) // (NS * ZR)) * (NS * ZR)
  PN = ((PN + _R - 1) // _R) * _R
  x = jnp.pad(x, ((0, PN - N), (0, 0)))
  batch = jnp.pad(batch, (0, PN - N), constant_values=G)

  d = _sc_deg(dst, PN)
  p = _sc_agg(x, src, dst)
  h, dinv = _tc_layer(p, p, d, d, x, W1l, W1r, b1.reshape(1, H),
                      first=True)
  for Wl, b, Wr in ((W2l, b2, W2r), (W3l, b3, W3r), (W4l, b4, W4r)):
    p = _sc_agg(h, src, dst)
    h, = _tc_layer(p, p, dinv, dinv, h, Wl, Wr, b.reshape(1, H),
                   first=False)
  batch3d = batch.reshape(PN // _R, 1, _R)
  root2d = root_node_idx.reshape(G, 1)
  return _tc_final(h, batch3d, root2d, Wc[:H], Wc[H:], bc.reshape(1, -1))


# speedup vs baseline: 8.2599x; 1.0678x over previous
import functools

import jax
import jax.numpy as jnp
from jax import lax
from jax.experimental import pallas as pl
from jax.experimental.pallas import tpu as pltpu
from jax.experimental.pallas import tpu_sc as plsc

NC = 2
NS = 16
CH = 80
ZR = 40
DW = 128
_R = 2048


def _sc_agg(h, src, dst):
  N, D = h.shape
  E = src.shape[0]
  assert E % (NC * NS) == 0
  epw = E // (NC * NS)
  assert epw % CH == 0 and epw % 8 == 0
  n_chunks = epw // CH
  assert N % NS == 0
  stripe = N // NS
  assert stripe % ZR == 0 and stripe % 8 == 0

  nq, tail = divmod(n_chunks, 4)
  mesh = plsc.VectorSubcoreMesh(core_axis_name="c", subcore_axis_name="s")

  @functools.partial(
      pl.kernel, mesh=mesh,
      out_type=jax.ShapeDtypeStruct((NC, N, D), jnp.float32),
      scratch_types=[
          pltpu.VMEM_SHARED((N, D), jnp.float32),
          pltpu.VMEM((ZR, D), jnp.float32),
      ] + [pltpu.VMEM((CH,), jnp.int32)] * 8
        + [pltpu.VMEM((CH, D), jnp.float32)] * 2
        + [pltpu.SemaphoreType.DMA] * 8)
  def agg_kernel(h_hbm, src_hbm, dst_hbm, p_hbm, acc, zb,
                 sb0, sb1, sb2, sb3, db0, db1, db2, db3, r0, r1,
                 sl0, sl1, sl2, sl3, sg0, sg1, ss0, ss1):
    sbs = (sb0, sb1, sb2, sb3)
    dbs = (db0, db1, db2, db3)
    rows = (r0, r1)
    sls = (sl0, sl1, sl2, sl3)
    sgs = (sg0, sg1)
    sss = (ss0, ss1)
    c = lax.axis_index("c")
    s = lax.axis_index("s")
    wid = s * NC + c

    @pl.loop(0, ZR)
    def _(i):
      @pl.loop(0, D, step=16)
      def _(j):
        zb.at[i, pl.ds(j, 16)][...] = jnp.zeros((16,), jnp.float32)

    @pl.loop(0, stripe // ZR)
    def _(t):
      pltpu.sync_copy(zb, acc.at[pl.ds(s * stripe + t * ZR, ZR)])

    plsc.subcore_barrier()

    base = wid * epw

    def start_idx_load(k, j):
      off = base + k * CH
      pltpu.async_copy(src_hbm.at[pl.ds(off, CH)], sbs[j], sls[j])
      pltpu.async_copy(dst_hbm.at[pl.ds(off, CH)], dbs[j], sls[j])

    def wait_idx(j):
      pltpu.make_async_copy(src_hbm.at[pl.ds(0, CH)], sbs[j], sls[j]).wait()
      pltpu.make_async_copy(dst_hbm.at[pl.ds(0, CH)], dbs[j], sls[j]).wait()

    def wait_scatter(b2, jprev):
      pltpu.make_async_copy(rows[b2], acc.at[dbs[jprev]], sss[b2]).wait()

    def chunk(k, j, kstat):
      b2 = j % 2
      jn = (j + 2) % 4
      if kstat is None:
        @pl.when(k >= 2)
        def _():
          wait_scatter(b2, jn)

        @pl.when(k + 2 < n_chunks)
        def _():
          start_idx_load(k + 2, jn)
      else:
        if kstat >= 2:
          wait_scatter(b2, jn)
        if kstat + 2 < n_chunks:
          start_idx_load(k + 2, jn)
      wait_idx(j)
      pltpu.async_copy(h_hbm.at[sbs[j]], rows[b2], sgs[b2])
      pltpu.make_async_copy(h_hbm.at[sbs[j]], rows[b2], sgs[b2]).wait()
      pltpu.async_copy(rows[b2], acc.at[dbs[j]], sss[b2], add=True)

    start_idx_load(0, 0)
    start_idx_load(1, 1)

    @pl.loop(0, nq)
    def _(q):
      k0 = q * 4
      for j in range(4):
        chunk(k0 + j, j, None)

    for t in range(tail):
      k = nq * 4 + t
      chunk(k, k % 4, k)

    if n_chunks >= 2:
      wait_scatter((n_chunks - 2) % 2, (n_chunks - 2) % 4)
    wait_scatter((n_chunks - 1) % 2, (n_chunks - 1) % 4)

    plsc.subcore_barrier()

    sl = pl.ds(s * stripe, stripe)
    pltpu.sync_copy(acc.at[sl], p_hbm.at[c, sl])

  return agg_kernel(h, src, dst)


def _sc_deg(dst, N):
  E = dst.shape[0]
  epw = E // (NC * NS)
  n_chunks = epw // CH
  stripe = N // NS

  mesh = plsc.VectorSubcoreMesh(core_axis_name="c", subcore_axis_name="s")

  nq, tail = divmod(n_chunks, 4)

  @functools.partial(
      pl.kernel, mesh=mesh,
      out_type=jax.ShapeDtypeStruct((NC, N, DW), jnp.float32),
      scratch_types=[
          pltpu.VMEM_SHARED((N, DW), jnp.float32),
          pltpu.VMEM((CH, DW), jnp.float32),
          pltpu.VMEM((ZR, DW), jnp.float32),
      ] + [pltpu.VMEM((CH,), jnp.int32)] * 4
        + [pltpu.SemaphoreType.DMA] * 6)
  def deg_kernel(dst_hbm, d_hbm, dacc, ones, zb,
                 db0, db1, db2, db3, sl0, sl1, sl2, sl3, ss0, ss1):
    dbs = (db0, db1, db2, db3)
    sls = (sl0, sl1, sl2, sl3)
    sss = (ss0, ss1)
    c = lax.axis_index("c")
    s = lax.axis_index("s")
    wid = s * NC + c

    @pl.loop(0, ZR)
    def _(i):
      @pl.loop(0, DW, step=16)
      def _(j):
        zb.at[i, pl.ds(j, 16)][...] = jnp.zeros((16,), jnp.float32)

    @pl.loop(0, CH)
    def _(i):
      @pl.loop(0, DW, step=16)
      def _(j):
        ones.at[i, pl.ds(j, 16)][...] = jnp.full((16,), 1.0, jnp.float32)

    @pl.loop(0, stripe // ZR)
    def _(t):
      pltpu.sync_copy(zb, dacc.at[pl.ds(s * stripe + t * ZR, ZR)])

    plsc.subcore_barrier()

    base = wid * epw

    def start_idx_load(k, j):
      pltpu.async_copy(dst_hbm.at[pl.ds(base + k * CH, CH)], dbs[j], sls[j])

    def wait_idx(j):
      pltpu.make_async_copy(dst_hbm.at[pl.ds(0, CH)], dbs[j], sls[j]).wait()

    def wait_scatter(b2, jprev):
      pltpu.make_async_copy(ones, dacc.at[dbs[jprev]], sss[b2]).wait()

    def chunk(k, j, kstat):
      b2 = j % 2
      jn = (j + 2) % 4
      if kstat is None:
        @pl.when(k >= 2)
        def _():
          wait_scatter(b2, jn)

        @pl.when(k + 2 < n_chunks)
        def _():
          start_idx_load(k + 2, jn)
      else:
        if kstat >= 2:
          wait_scatter(b2, jn)
        if kstat + 2 < n_chunks:
          start_idx_load(k + 2, jn)
      wait_idx(j)
      pltpu.async_copy(ones, dacc.at[dbs[j]], sss[b2], add=True)

    start_idx_load(0, 0)
    start_idx_load(1, 1)

    @pl.loop(0, nq)
    def _(q):
      k0 = q * 4
      for j in range(4):
        chunk(k0 + j, j, None)

    for t in range(tail):
      k = nq * 4 + t
      chunk(k, k % 4, k)

    if n_chunks >= 2:
      wait_scatter((n_chunks - 2) % 2, (n_chunks - 2) % 4)
    wait_scatter((n_chunks - 1) % 2, (n_chunks - 1) % 4)

    plsc.subcore_barrier()

    sl = pl.ds(s * stripe, stripe)
    pltpu.sync_copy(dacc.at[sl], d_hbm.at[c, sl])

  return deg_kernel(dst)


def _tc_layer(p0, p1, dscale0, dscale1, h, Wl, Wr, b, first):
  N, D = h.shape
  H = Wl.shape[1]
  grid = (N // _R,)

  def body(p0_r, p1_r, d0_r, d1_r, h_r, wl_r, wr_r, b_r, *outs):
    if first:
      o_r, dinv_r = outs
      deg = (d0_r[0] + d1_r[0])[:, 0:1]
      dinv = 1.0 / jnp.maximum(deg, 1.0)
      dinv_r[...] = dinv
    else:
      o_r, = outs
      dinv = d0_r[...]
    mean = (p0_r[0] + p1_r[0]) * dinv
    z = (jnp.dot(mean, wl_r[...], preferred_element_type=jnp.float32)
         + jnp.dot(h_r[...], wr_r[...], preferred_element_type=jnp.float32)
         + b_r[...])
    o_r[...] = jnp.maximum(z, 0.0)

  row = lambda i: (i, 0)
  fix = lambda i: (0, 0)
  lo3 = lambda i: (0, i, 0)
  hi3 = lambda i: (1, i, 0)
  if first:
    d_specs = [pl.BlockSpec((1, _R, DW), lo3), pl.BlockSpec((1, _R, DW), hi3)]
  else:
    d_specs = [pl.BlockSpec((_R, 1), row), pl.BlockSpec((_R, 1), row)]
  out_specs = [pl.BlockSpec((_R, H), row)]
  out_shape = [jax.ShapeDtypeStruct((N, H), jnp.float32)]
  if first:
    out_specs.append(pl.BlockSpec((_R, 1), row))
    out_shape.append(jax.ShapeDtypeStruct((N, 1), jnp.float32))
  res = pl.pallas_call(
      body,
      grid=grid,
      in_specs=[
          pl.BlockSpec((1, _R, D), lo3),
          pl.BlockSpec((1, _R, D), hi3),
          d_specs[0],
          d_specs[1],
          pl.BlockSpec((_R, D), row),
          pl.BlockSpec((D, H), fix),
          pl.BlockSpec((D, H), fix),
          pl.BlockSpec((1, H), fix),
      ],
      out_specs=out_specs,
      out_shape=out_shape,
  )(p0, p1, dscale0, dscale1, h, Wl, Wr, b)
  return res


def _tc_final(h, batch3d, root2d, Wc1, Wc2, bc):
  N, H = h.shape
  G = root2d.shape[0]
  OUT = Wc1.shape[1]
  nblk = N // _R

  def body(h_r, b_r, r_r, wc1_r, wc2_r, bc_r, o_r, racc, sacc, cacc):
    i = pl.program_id(0)

    @pl.when(i == 0)
    def _():
      racc[...] = jnp.zeros_like(racc)
      sacc[...] = jnp.zeros_like(sacc)
      cacc[...] = jnp.zeros_like(cacc)

    rows = lax.broadcasted_iota(jnp.int32, (G, _R), 1) + i * _R
    rmask = (r_r[...] == rows).astype(jnp.float32)
    g_iota = lax.broadcasted_iota(jnp.int32, (G, _R), 0)
    bmask = (b_r[0] == g_iota).astype(jnp.float32)

    hb = h_r[...]
    racc[...] += jnp.dot(rmask, hb, preferred_element_type=jnp.float32)
    sacc[...] += jnp.dot(bmask, hb, preferred_element_type=jnp.float32)
    cacc[...] = cacc[...] + jnp.sum(bmask, axis=1, keepdims=True)

    @pl.when(i == nblk - 1)
    def _():
      ge = sacc[...] / jnp.maximum(cacc[...], 1.0)
      o_r[...] = (jnp.dot(racc[...], wc1_r[...],
                          preferred_element_type=jnp.float32)
                  + jnp.dot(ge, wc2_r[...],
                            preferred_element_type=jnp.float32)
                  + bc_r[...])

  return pl.pallas_call(
      body,
      grid=(nblk,),
      in_specs=[
          pl.BlockSpec((_R, H), lambda i: (i, 0)),
          pl.BlockSpec((1, 1, _R), lambda i: (i, 0, 0)),
          pl.BlockSpec((G, 1), lambda i: (0, 0)),
          pl.BlockSpec((H, OUT), lambda i: (0, 0)),
          pl.BlockSpec((H, OUT), lambda i: (0, 0)),
          pl.BlockSpec((1, OUT), lambda i: (0, 0)),
      ],
      out_specs=pl.BlockSpec((G, OUT), lambda i: (0, 0)),
      out_shape=jax.ShapeDtypeStruct((G, OUT), jnp.float32),
      scratch_shapes=[
          pltpu.VMEM((G, H), jnp.float32),
          pltpu.VMEM((G, H), jnp.float32),
          pltpu.VMEM((G, 128), jnp.float32),
      ],
  )(h, batch3d, root2d, Wc1, Wc2, bc)


def kernel(x, edge_index, root_node_idx, batch,
           W1l, b1, W1r, W2l, b2, W2r, W3l, b3, W3r, W4l, b4, W4r, Wc, bc):
  N, D = x.shape
  H = W1l.shape[1]
  G = root_node_idx.shape[0]
  src = edge_index[0]
  dst = edge_index[1]

  PN = ((N + NS * ZR - 1) // (NS * ZR)) * (NS * ZR)
  PN = ((PN + _R - 1) // _R) * _R
  x = jnp.pad(x, ((0, PN - N), (0, 0)))
  batch = jnp.pad(batch, (0, PN - N), constant_values=G)

  d = _sc_deg(dst, PN)
  p = _sc_agg(x, src, dst)
  h, dinv = _tc_layer(p, p, d, d, x, W1l, W1r, b1.reshape(1, H),
                      first=True)
  for Wl, b, Wr in ((W2l, b2, W2r), (W3l, b3, W3r), (W4l, b4, W4r)):
    p = _sc_agg(h, src, dst)
    h, = _tc_layer(p, p, dinv, dinv, h, Wl, Wr, b.reshape(1, H),
                   first=False)
  batch3d = batch.reshape(PN // _R, 1, _R)
  root2d = root_node_idx.reshape(G, 1)
  return _tc_final(h, batch3d, root2d, Wc[:H], Wc[H:], bc.reshape(1, -1))
